# baseline (speedup 1.0000x reference)
import functools
import math

import jax
import jax.numpy as jnp
import numpy as np
from jax import lax
from jax.experimental import pallas as pl
from jax.experimental.pallas import tpu as pltpu
from jax.experimental.pallas import tpu_sc as plsc

_LANES = 16
_BB = 128
_NBUF = 4


@functools.lru_cache(maxsize=None)
def _make_kernel(B, S, D, scale):
    info = plsc.get_sparse_core_info()
    NC, NS = info.num_cores, info.num_subcores
    NW = NC * NS
    nb = B // _BB
    units = S * nb
    per_w = units // NW
    assert per_w * NW == units and per_w % _NBUF == 0
    ndb = D // 8
    mesh = plsc.VectorSubcoreMesh(core_axis_name="c", subcore_axis_name="s")

    @functools.partial(
        pl.kernel,
        mesh=mesh,
        compiler_params=pltpu.CompilerParams(
            use_tc_tiling_on_sc=False, needs_layout_passes=False),
        out_type=jax.ShapeDtypeStruct((S, ndb, nb, 8, _BB), jnp.float32),
        scratch_types=[
            pltpu.VMEM((per_w * _BB,), jnp.int32),
            pltpu.VMEM((S * D,), jnp.float32),
            pltpu.VMEM((_NBUF, _BB, D), jnp.float32),
            pltpu.VMEM((_NBUF, ndb, 8, _BB), jnp.float32),
        ]
        + [pltpu.SemaphoreType.DMA] * (2 * _NBUF),
    )
    def k(xt_hbm, table_hbm, pos_hbm, out_hbm, idx_v, pos_v, rows, outt,
          *sems):
        gsem = sems[:_NBUF]
        ssem = sems[_NBUF:]
        wid = lax.axis_index("s") * NC + lax.axis_index("c")
        ubase = wid * per_w
        pltpu.sync_copy(xt_hbm.at[pl.ds(ubase * _BB, per_w * _BB)], idx_v)
        pltpu.sync_copy(pos_hbm, pos_v)
        lanes = lax.iota(jnp.int32, _LANES)

        def fire_gather(t, b):
            off = pl.multiple_of(t * _BB, _BB)
            pltpu.async_copy(
                table_hbm.at[idx_v.at[pl.ds(off, _BB)]], rows.at[b], gsem[b])

        def drain_gather(b):
            pltpu.make_async_copy(
                table_hbm.at[pl.ds(0, _BB)], rows.at[b], gsem[b]).wait()

        def drain_store(b):
            for dblk in range(ndb):
                pltpu.make_async_copy(
                    outt.at[b, dblk], out_hbm.at[0, dblk, 0], ssem[b]).wait()

        def compute(t, b):
            u = ubase + t
            s = u // nb
            pbase = s * D

            def dbody(d, carry):
                pidx = jnp.broadcast_to(pbase + d, (_LANES,))
                pvec = plsc.load_gather(pos_v, [pidx])
                dsplat = jnp.broadcast_to(d, (_LANES,))
                dblk = d // 8
                din = d % 8
                for g in range(_BB // _LANES):
                    ridx = lanes + g * _LANES
                    vec = plsc.load_gather(rows.at[b], [ridx, dsplat])
                    outt[b, dblk, din, pl.ds(g * _LANES, _LANES)] = (
                        vec * scale + pvec)
                return carry

            lax.fori_loop(0, D, dbody, 0)

        def fire_store(t, b):
            u = ubase + t
            s = u // nb
            bblk = u % nb
            for dblk in range(ndb):
                pltpu.async_copy(
                    outt.at[b, dblk], out_hbm.at[s, dblk, bblk], ssem[b])

        fire_gather(0, 0)
        fire_gather(1, 1)

        def loop_body(p, carry):
            for bb in range(_NBUF):
                t = p * _NBUF + bb
                drain_gather(bb)

                @pl.when(t >= _NBUF)
                def _():
                    drain_store(bb)

                compute(t, bb)
                fire_store(t, bb)
                t2 = t + 2

                @pl.when(t2 < per_w)
                def _():
                    fire_gather(t2, (bb + 2) % _NBUF)
            return carry

        lax.fori_loop(0, per_w // _NBUF, loop_body, 0)
        for bb in range(_NBUF):
            drain_store(bb)

    return k


def kernel(x, table, pos_encoding, training=False):
    B, S = x.shape
    D = table.shape[1]
    scale = float(np.float32(math.sqrt(D)))
    xt = x.T.reshape(-1)
    posf = pos_encoding[0, :S, :].astype(jnp.float32).reshape(-1)
    k = _make_kernel(B, S, D, scale)
    out5 = k(xt, table, posf)
    return out5.transpose(2, 4, 0, 1, 3).reshape(B, S, D)


# speedup vs baseline: 1.6692x; 1.6692x over previous
import functools
import math

import jax
import jax.numpy as jnp
import numpy as np
from jax import lax
from jax.experimental import pallas as pl
from jax.experimental.pallas import tpu as pltpu
from jax.experimental.pallas import tpu_sc as plsc

_LANES = 16
_BB = 128
_NBUF = 4


@functools.lru_cache(maxsize=None)
def _make_kernel(B, S, D, scale):
    info = plsc.get_sparse_core_info()
    NC, NS = info.num_cores, info.num_subcores
    NW = NC * NS
    nb = B // _BB
    units = S * nb
    per_w = units // NW
    assert per_w * NW == units and per_w % _NBUF == 0
    ndb = D // 8
    mesh = plsc.VectorSubcoreMesh(core_axis_name="c", subcore_axis_name="s")

    @functools.partial(
        pl.kernel,
        mesh=mesh,
        compiler_params=pltpu.CompilerParams(
            use_tc_tiling_on_sc=False, needs_layout_passes=False),
        out_type=jax.ShapeDtypeStruct((S, ndb, nb, 8, _BB), jnp.float32),
        scratch_types=[
            pltpu.VMEM((per_w * _BB,), jnp.int32),
            pltpu.VMEM((S * D,), jnp.float32),
            pltpu.VMEM((_NBUF, _BB, D), jnp.float32),
            pltpu.VMEM((_NBUF, ndb, 8, _BB + 1), jnp.float32),
        ]
        + [pltpu.SemaphoreType.DMA] * (2 * _NBUF),
    )
    def k(xt_hbm, table_hbm, pos_hbm, out_hbm, idx_v, pos_v, rows, outt,
          *sems):
        gsem = sems[:_NBUF]
        ssem = sems[_NBUF:]
        wid = lax.axis_index("s") * NC + lax.axis_index("c")
        ubase = wid * per_w
        pltpu.sync_copy(xt_hbm.at[pl.ds(ubase * _BB, per_w * _BB)], idx_v)
        pltpu.sync_copy(pos_hbm, pos_v)
        lanes = lax.iota(jnp.int32, _LANES)

        def fire_gather(t, b):
            off = pl.multiple_of(t * _BB, _BB)
            pltpu.async_copy(
                table_hbm.at[idx_v.at[pl.ds(off, _BB)]], rows.at[b], gsem[b])

        def drain_gather(b):
            pltpu.make_async_copy(
                table_hbm.at[pl.ds(0, _BB)], rows.at[b], gsem[b]).wait()

        def drain_store(b):
            for dblk in range(ndb):
                pltpu.make_async_copy(
                    outt.at[b, dblk, :, pl.ds(0, _BB)],
                    out_hbm.at[0, dblk, 0], ssem[b]).wait()

        ncol = D // _LANES
        dblk_idx = [(c * _LANES + lanes) // 8 for c in range(ncol)]
        din_idx = [(c * _LANES + lanes) % 8 for c in range(ncol)]

        def compute(t, b):
            u = ubase + t
            s = u // nb
            pbase = s * D
            pv = [pos_v[pl.ds(pbase + c * _LANES, _LANES)]
                  for c in range(ncol)]

            def rbody(r, carry):
                rv = jnp.broadcast_to(r, (_LANES,))
                for c in range(ncol):
                    vec = rows[b, r, pl.ds(c * _LANES, _LANES)]
                    plsc.store_scatter(
                        outt.at[b], [dblk_idx[c], din_idx[c], rv],
                        vec * scale + pv[c])
                return carry

            lax.fori_loop(0, _BB, rbody, 0)

        def fire_store(t, b):
            u = ubase + t
            s = u // nb
            bblk = u % nb
            for dblk in range(ndb):
                pltpu.async_copy(
                    outt.at[b, dblk, :, pl.ds(0, _BB)],
                    out_hbm.at[s, dblk, bblk], ssem[b])

        fire_gather(0, 0)
        fire_gather(1, 1)

        def loop_body(p, carry):
            for bb in range(_NBUF):
                t = p * _NBUF + bb
                drain_gather(bb)

                @pl.when(t >= _NBUF)
                def _():
                    drain_store(bb)

                compute(t, bb)
                fire_store(t, bb)
                t2 = t + 2

                @pl.when(t2 < per_w)
                def _():
                    fire_gather(t2, (bb + 2) % _NBUF)
            return carry

        lax.fori_loop(0, per_w // _NBUF, loop_body, 0)
        for bb in range(_NBUF):
            drain_store(bb)

    return k


def kernel(x, table, pos_encoding, training=False):
    B, S = x.shape
    D = table.shape[1]
    scale = float(np.float32(math.sqrt(D)))
    xt = x.T.reshape(-1)
    posf = pos_encoding[0, :S, :].astype(jnp.float32).reshape(-1)
    k = _make_kernel(B, S, D, scale)
    out5 = k(xt, table, posf)
    return out5.transpose(2, 4, 0, 1, 3).reshape(B, S, D)


# speedup vs baseline: 1.6772x; 1.0048x over previous
import functools
import math

import jax
import jax.numpy as jnp
import numpy as np
from jax import lax
from jax.experimental import pallas as pl
from jax.experimental.pallas import tpu as pltpu
from jax.experimental.pallas import tpu_sc as plsc

_LANES = 16
_BB = 128
_NBUF = 4


@functools.lru_cache(maxsize=None)
def _make_kernel(B, S, D, scale):
    info = plsc.get_sparse_core_info()
    NC, NS = info.num_cores, info.num_subcores
    NW = NC * NS
    nb = B // _BB
    units = S * nb
    per_w = units // NW
    assert per_w * NW == units and per_w % _NBUF == 0
    ndb = D // 8
    mesh = plsc.VectorSubcoreMesh(core_axis_name="c", subcore_axis_name="s")

    @functools.partial(
        pl.kernel,
        mesh=mesh,
        compiler_params=pltpu.CompilerParams(
            use_tc_tiling_on_sc=False, needs_layout_passes=False),
        out_type=jax.ShapeDtypeStruct((S, ndb, nb, 8, _BB), jnp.float32),
        scratch_types=[
            pltpu.VMEM((per_w * _BB,), jnp.int32),
            pltpu.VMEM((S * D,), jnp.float32),
            pltpu.VMEM((_NBUF, _BB, D), jnp.float32),
            pltpu.VMEM((_NBUF, ndb, 8, _BB + 1), jnp.float32),
        ]
        + [pltpu.SemaphoreType.DMA] * (2 * _NBUF),
    )
    def k(xt_hbm, table_hbm, pos_hbm, out_hbm, idx_v, pos_v, rows, outt,
          *sems):
        gsem = sems[:_NBUF]
        ssem = sems[_NBUF:]
        wid = lax.axis_index("s") * NC + lax.axis_index("c")
        ubase = wid * per_w
        pltpu.sync_copy(xt_hbm.at[pl.ds(ubase * _BB, per_w * _BB)], idx_v)
        pltpu.sync_copy(pos_hbm, pos_v)
        lanes = lax.iota(jnp.int32, _LANES)

        def fire_gather(t, b):
            off = pl.multiple_of(t * _BB, _BB)
            pltpu.async_copy(
                table_hbm.at[idx_v.at[pl.ds(off, _BB)]], rows.at[b], gsem[b])

        def drain_gather(b):
            pltpu.make_async_copy(
                table_hbm.at[pl.ds(0, _BB)], rows.at[b], gsem[b]).wait()

        def drain_store(b):
            for dblk in range(ndb):
                pltpu.make_async_copy(
                    outt.at[b, dblk, :, pl.ds(0, _BB)],
                    out_hbm.at[0, dblk, 0], ssem[b]).wait()

        ncol = D // _LANES
        dblk_idx = [(c * _LANES + lanes) // 8 for c in range(ncol)]
        din_idx = [(c * _LANES + lanes) % 8 for c in range(ncol)]

        def compute(t, b):
            u = ubase + t
            s = u // nb
            pbase = s * D
            pv = [pos_v[pl.ds(pbase + c * _LANES, _LANES)]
                  for c in range(ncol)]

            def rbody(q, carry):
                r0 = q * 4
                for dr in range(4):
                    r = r0 + dr
                    rv = jnp.broadcast_to(r, (_LANES,))
                    for c in range(ncol):
                        vec = rows[b, r, pl.ds(c * _LANES, _LANES)]
                        plsc.store_scatter(
                            outt.at[b], [dblk_idx[c], din_idx[c], rv],
                            vec * scale + pv[c])
                return carry

            lax.fori_loop(0, _BB // 4, rbody, 0)

        def fire_store(t, b):
            u = ubase + t
            s = u // nb
            bblk = u % nb
            for dblk in range(ndb):
                pltpu.async_copy(
                    outt.at[b, dblk, :, pl.ds(0, _BB)],
                    out_hbm.at[s, dblk, bblk], ssem[b])

        fire_gather(0, 0)
        fire_gather(1, 1)

        def loop_body(p, carry):
            for bb in range(_NBUF):
                t = p * _NBUF + bb
                drain_gather(bb)

                @pl.when(t >= _NBUF)
                def _():
                    drain_store(bb)

                compute(t, bb)
                fire_store(t, bb)
                t2 = t + 2

                @pl.when(t2 < per_w)
                def _():
                    fire_gather(t2, (bb + 2) % _NBUF)
            return carry

        lax.fori_loop(0, per_w // _NBUF, loop_body, 0)
        for bb in range(_NBUF):
            drain_store(bb)

    return k


def kernel(x, table, pos_encoding, training=False):
    B, S = x.shape
    D = table.shape[1]
    scale = float(np.float32(math.sqrt(D)))
    xt = x.T.reshape(-1)
    posf = pos_encoding[0, :S, :].astype(jnp.float32).reshape(-1)
    k = _make_kernel(B, S, D, scale)
    out5 = k(xt, table, posf)
    return out5.transpose(2, 4, 0, 1, 3).reshape(B, S, D)


# speedup vs baseline: 2.5634x; 1.5284x over previous
import functools
import math

import jax
import jax.numpy as jnp
import numpy as np
from jax import lax
from jax.experimental import pallas as pl
from jax.experimental.pallas import tpu as pltpu
from jax.experimental.pallas import tpu_sc as plsc

_LANES = 16
_BB = 128
_NBUF = 4


@functools.lru_cache(maxsize=None)
def _make_kernel(B, S, D, scale):
    info = plsc.get_sparse_core_info()
    NC, NS = info.num_cores, info.num_subcores
    NW = NC * NS
    nb = B // _BB
    units = S * nb
    per_w = units // NW
    assert per_w * NW == units and per_w % _NBUF == 0
    ndb = D // 8
    mesh = plsc.VectorSubcoreMesh(core_axis_name="c", subcore_axis_name="s")

    @functools.partial(
        pl.kernel,
        mesh=mesh,
        compiler_params=pltpu.CompilerParams(
            use_tc_tiling_on_sc=False, needs_layout_passes=False),
        out_type=jax.ShapeDtypeStruct((S, ndb, nb, 8, _BB), jnp.float32),
        scratch_types=[
            pltpu.VMEM((per_w * _BB,), jnp.int32),
            pltpu.VMEM((S * D,), jnp.float32),
            pltpu.VMEM((_NBUF, _BB, D), jnp.float32),
            pltpu.VMEM((_NBUF, ndb, 8, _BB + 1), jnp.float32),
        ]
        + [pltpu.SemaphoreType.DMA] * (2 * _NBUF),
    )
    def k(xt_hbm, table_hbm, pos_hbm, out_hbm, idx_v, pos_v, rows, outt,
          *sems):
        gsem = sems[:_NBUF]
        ssem = sems[_NBUF:]
        wid = lax.axis_index("s") * NC + lax.axis_index("c")
        ubase = wid * per_w
        pltpu.sync_copy(xt_hbm.at[pl.ds(ubase * _BB, per_w * _BB)], idx_v)
        pltpu.sync_copy(pos_hbm, pos_v)
        lanes = lax.iota(jnp.int32, _LANES)

        def fire_gather(t, b):
            off = pl.multiple_of(t * _BB, _BB)
            pltpu.async_copy(
                table_hbm.at[idx_v.at[pl.ds(off, _BB)]], rows.at[b], gsem[b])

        def drain_gather(b):
            pltpu.make_async_copy(
                table_hbm.at[pl.ds(0, _BB)], rows.at[b], gsem[b]).wait()

        def drain_store(b):
            for dblk in range(ndb):
                pltpu.make_async_copy(
                    outt.at[b, dblk, :, pl.ds(0, _BB)],
                    out_hbm.at[0, dblk, 0], ssem[b]).wait()

        ncol = D // _LANES
        dblk_idx = [(c * _LANES + lanes) // 8 for c in range(ncol)]
        din_idx = [(c * _LANES + lanes) % 8 for c in range(ncol)]

        def compute(t, b):
            u = ubase + t
            s = u // nb
            pbase = s * D
            pv = [pos_v[pl.ds(pbase + c * _LANES, _LANES)]
                  for c in range(ncol)]

            @plsc.parallel_loop(0, _BB, step=1, unroll=8)
            def _(r):
                rv = jnp.broadcast_to(r, (_LANES,))
                for c in range(ncol):
                    vec = rows[b, r, pl.ds(c * _LANES, _LANES)]
                    plsc.store_scatter(
                        outt.at[b], [dblk_idx[c], din_idx[c], rv],
                        vec * scale + pv[c])

        def fire_store(t, b):
            u = ubase + t
            s = u // nb
            bblk = u % nb
            for dblk in range(ndb):
                pltpu.async_copy(
                    outt.at[b, dblk, :, pl.ds(0, _BB)],
                    out_hbm.at[s, dblk, bblk], ssem[b])

        fire_gather(0, 0)
        fire_gather(1, 1)

        def loop_body(p, carry):
            for bb in range(_NBUF):
                t = p * _NBUF + bb
                drain_gather(bb)

                @pl.when(t >= _NBUF)
                def _():
                    drain_store(bb)

                compute(t, bb)
                fire_store(t, bb)
                t2 = t + 2

                @pl.when(t2 < per_w)
                def _():
                    fire_gather(t2, (bb + 2) % _NBUF)
            return carry

        lax.fori_loop(0, per_w // _NBUF, loop_body, 0)
        for bb in range(_NBUF):
            drain_store(bb)

    return k


def kernel(x, table, pos_encoding, training=False):
    B, S = x.shape
    D = table.shape[1]
    scale = float(np.float32(math.sqrt(D)))
    xt = x.T.reshape(-1)
    posf = pos_encoding[0, :S, :].astype(jnp.float32).reshape(-1)
    k = _make_kernel(B, S, D, scale)
    out5 = k(xt, table, posf)
    return out5.transpose(2, 4, 0, 1, 3).reshape(B, S, D)


# speedup vs baseline: 2.6024x; 1.0152x over previous
import functools
import math

import jax
import jax.numpy as jnp
import numpy as np
from jax import lax
from jax.experimental import pallas as pl
from jax.experimental.pallas import tpu as pltpu
from jax.experimental.pallas import tpu_sc as plsc

_LANES = 16
_BB = 128
_NBUF = 4
_NOUT = 2
_TW = 64


@functools.lru_cache(maxsize=None)
def _make_kernel(B, S, D, scale):
    info = plsc.get_sparse_core_info()
    NC, NS = info.num_cores, info.num_subcores
    NW = NC * NS
    nb = B // _BB
    units = S * nb
    per_w = units // NW
    assert per_w * NW == units and per_w % _NBUF == 0
    ndb = D // 8
    mesh = plsc.VectorSubcoreMesh(core_axis_name="c", subcore_axis_name="s")

    @functools.partial(
        pl.kernel,
        mesh=mesh,
        compiler_params=pltpu.CompilerParams(
            use_tc_tiling_on_sc=False, needs_layout_passes=False),
        out_type=jax.ShapeDtypeStruct((S, ndb, nb, 8, _BB), jnp.float32),
        scratch_types=[
            pltpu.VMEM((per_w * _BB,), jnp.int32),
            pltpu.VMEM((S * D,), jnp.float32),
            pltpu.VMEM((_NBUF, _BB, _TW), jnp.float32),
            pltpu.VMEM((_NOUT, ndb, 8, _BB + 1), jnp.float32),
        ]
        + [pltpu.SemaphoreType.DMA] * (_NBUF + _NOUT),
    )
    def k(xt_hbm, table_hbm, pos_hbm, out_hbm, idx_v, pos_v, rows, outt,
          *sems):
        gsem = sems[:_NBUF]
        ssem = sems[_NBUF:]
        wid = lax.axis_index("s") * NC + lax.axis_index("c")
        ubase = wid * per_w
        pltpu.sync_copy(xt_hbm.at[pl.ds(ubase * _BB, per_w * _BB)], idx_v)
        pltpu.sync_copy(pos_hbm, pos_v)
        lanes = lax.iota(jnp.int32, _LANES)

        def fire_gather(t, b):
            off = pl.multiple_of(t * _BB, _BB)
            pltpu.async_copy(
                table_hbm.at[idx_v.at[pl.ds(off, _BB)]], rows.at[b], gsem[b])

        def drain_gather(b):
            pltpu.make_async_copy(
                table_hbm.at[pl.ds(0, _BB)], rows.at[b], gsem[b]).wait()

        def drain_store(b):
            for dblk in range(ndb):
                pltpu.make_async_copy(
                    outt.at[b, dblk, :, pl.ds(0, _BB)],
                    out_hbm.at[0, dblk, 0], ssem[b]).wait()

        ncol = D // _LANES
        dblk_idx = [(c * _LANES + lanes) // 8 for c in range(ncol)]
        din_idx = [(c * _LANES + lanes) % 8 for c in range(ncol)]

        def compute(t, b, ob):
            u = ubase + t
            s = u // nb
            pbase = s * D
            pv = [pos_v[pl.ds(pbase + c * _LANES, _LANES)]
                  for c in range(ncol)]

            @plsc.parallel_loop(0, _BB, step=1, unroll=16)
            def _(r):
                rv = jnp.broadcast_to(r, (_LANES,))
                for c in range(ncol):
                    vec = rows[b, r, pl.ds(c * _LANES, _LANES)]
                    plsc.store_scatter(
                        outt.at[ob], [dblk_idx[c], din_idx[c], rv],
                        vec * scale + pv[c])

        def fire_store(t, ob):
            u = ubase + t
            s = u // nb
            bblk = u % nb
            for dblk in range(ndb):
                pltpu.async_copy(
                    outt.at[ob, dblk, :, pl.ds(0, _BB)],
                    out_hbm.at[s, dblk, bblk], ssem[ob])

        fire_gather(0, 0)
        fire_gather(1, 1)
        fire_gather(2, 2)

        def loop_body(p, carry):
            for bb in range(_NBUF):
                t = p * _NBUF + bb
                ob = bb % _NOUT
                drain_gather(bb)

                @pl.when(t >= _NOUT)
                def _():
                    drain_store(ob)

                compute(t, bb, ob)
                fire_store(t, ob)
                t2 = t + 3

                @pl.when(t2 < per_w)
                def _():
                    fire_gather(t2, (bb + 3) % _NBUF)
            return carry

        lax.fori_loop(0, per_w // _NBUF, loop_body, 0)
        for ob in range(_NOUT):
            drain_store(ob)

    return k


def kernel(x, table, pos_encoding, training=False):
    B, S = x.shape
    D = table.shape[1]
    scale = float(np.float32(math.sqrt(D)))
    xt = x.T.reshape(-1)
    posf = pos_encoding[0, :S, :].astype(jnp.float32).reshape(-1)
    k = _make_kernel(B, S, D, scale)
    out5 = k(xt, table, posf)
    return out5.transpose(2, 4, 0, 1, 3).reshape(B, S, D)
